# baseline (speedup 1.0000x reference)
import functools

import jax
import jax.numpy as jnp
from jax import lax
from jax.experimental import pallas as pl


def _vq_body(ze_ref, w_ref, out_ref, *, block_rows, n_codes):
    zeb = ze_ref[...]
    w = w_ref[...]
    ze2 = jnp.sum(zeb * zeb, axis=1, keepdims=True)
    w2 = lax.dot_general(
        jnp.ones((1, w.shape[1]), jnp.float32), w * w,
        (((1,), (1,)), ((), ())), precision=lax.Precision.HIGHEST,
        preferred_element_type=jnp.float32)
    mm = lax.dot_general(zeb, w, (((1,), (1,)), ((), ())),
                         preferred_element_type=jnp.float32)
    dist = jnp.sqrt(jnp.maximum(ze2 + w2 - 2.0 * mm, 0.0))
    m = jnp.min(dist, axis=1, keepdims=True)
    iota = lax.broadcasted_iota(jnp.int32, (block_rows, n_codes), 1)
    idx = jnp.min(jnp.where(dist == m, iota, n_codes), axis=1,
                  keepdims=True)
    onehot = (iota == idx).astype(jnp.bfloat16)
    w_hi = w.astype(jnp.bfloat16)
    w_lo = (w - w_hi.astype(jnp.float32)).astype(jnp.bfloat16)
    dims = (((1,), (0,)), ((), ()))
    out_ref[...] = (
        lax.dot_general(onehot, w_hi, dims,
                        preferred_element_type=jnp.float32)
        + lax.dot_general(onehot, w_lo, dims,
                          preferred_element_type=jnp.float32))


def kernel(ze, emb_w):
    B, N, D = ze.shape
    K = emb_w.shape[0]
    M = B * N
    BR = 1152
    zef = ze.reshape(M, D)
    out = pl.pallas_call(
        functools.partial(_vq_body, block_rows=BR, n_codes=K),
        grid=(M // BR,),
        in_specs=[pl.BlockSpec((BR, D), lambda i: (i, 0)),
                  pl.BlockSpec((K, D), lambda i: (0, 0))],
        out_specs=pl.BlockSpec((BR, D), lambda i: (i, 0)),
        out_shape=jax.ShapeDtypeStruct((M, D), jnp.float32),
    )(zef, emb_w)
    return out.reshape(B, N, D)


# speedup vs baseline: 1.0963x; 1.0963x over previous
import functools

import jax
import jax.numpy as jnp
from jax import lax
from jax.experimental import pallas as pl


def _vq_body(ze_ref, w_ref, out_ref, *, block_rows, n_codes):
    zeb = ze_ref[...]
    w = w_ref[...]
    ze2 = jnp.sum(zeb * zeb, axis=1, keepdims=True)
    w2 = lax.dot_general(
        jnp.ones((1, w.shape[1]), jnp.float32), w * w,
        (((1,), (1,)), ((), ())), precision=lax.Precision.HIGHEST,
        preferred_element_type=jnp.float32)
    mm2 = lax.dot_general(2.0 * zeb, w, (((1,), (1,)), ((), ())),
                          preferred_element_type=jnp.float32)
    dist = jnp.sqrt(jnp.maximum(ze2 + w2 - mm2, 0.0))
    m = jnp.min(dist, axis=1, keepdims=True)
    iota = lax.broadcasted_iota(jnp.int32, (block_rows, n_codes), 1)
    idx = jnp.min(jnp.where(dist == m, iota, n_codes), axis=1,
                  keepdims=True)
    onehot = (iota == idx).astype(jnp.bfloat16)
    w_hi = w.astype(jnp.bfloat16)
    w_lo = (w - w_hi.astype(jnp.float32)).astype(jnp.bfloat16)
    w_cat = jnp.concatenate((w_hi, w_lo), axis=1)
    dims = (((1,), (0,)), ((), ()))
    both = lax.dot_general(onehot, w_cat, dims,
                           preferred_element_type=jnp.float32)
    D = w.shape[1]
    out_ref[...] = both[:, :D] + both[:, D:]


def kernel(ze, emb_w):
    B, N, D = ze.shape
    K = emb_w.shape[0]
    M = B * N
    BR = 1152
    zef = ze.reshape(M, D)
    out = pl.pallas_call(
        functools.partial(_vq_body, block_rows=BR, n_codes=K),
        grid=(M // BR,),
        in_specs=[pl.BlockSpec((BR, D), lambda i: (i, 0)),
                  pl.BlockSpec((K, D), lambda i: (0, 0))],
        out_specs=pl.BlockSpec((BR, D), lambda i: (i, 0)),
        out_shape=jax.ShapeDtypeStruct((M, D), jnp.float32),
    )(zef, emb_w)
    return out.reshape(B, N, D)


# speedup vs baseline: 1.3489x; 1.2304x over previous
import functools

import jax
import jax.numpy as jnp
from jax import lax
from jax.experimental import pallas as pl


def _vq_body(ze_ref, w_ref, out_ref, *, block_rows, n_codes, n_sub):
    w = w_ref[...]
    D = w.shape[1]
    w2 = lax.dot_general(
        jnp.ones((1, D), jnp.float32), w * w,
        (((1,), (1,)), ((), ())), precision=lax.Precision.HIGHEST,
        preferred_element_type=jnp.float32)
    w_hi = w.astype(jnp.bfloat16)
    w_lo = (w - w_hi.astype(jnp.float32)).astype(jnp.bfloat16)
    w_cat = jnp.concatenate((w_hi, w_lo), axis=1)
    SR = block_rows // n_sub
    iota = lax.broadcasted_iota(jnp.int32, (SR, n_codes), 1)
    for s in range(n_sub):
        zeb = ze_ref[s * SR:(s + 1) * SR, :]
        ze2 = jnp.sum(zeb * zeb, axis=1, keepdims=True)
        mm2 = lax.dot_general(2.0 * zeb, w, (((1,), (1,)), ((), ())),
                              preferred_element_type=jnp.float32)
        dist = jnp.sqrt(jnp.maximum(ze2 + w2 - mm2, 0.0))
        m = jnp.min(dist, axis=1, keepdims=True)
        idx = jnp.min(jnp.where(dist == m, iota, n_codes), axis=1,
                      keepdims=True)
        onehot = (iota == idx).astype(jnp.bfloat16)
        both = lax.dot_general(onehot, w_cat, (((1,), (0,)), ((), ())),
                               preferred_element_type=jnp.float32)
        out_ref[s * SR:(s + 1) * SR, :] = both[:, :D] + both[:, D:]


def kernel(ze, emb_w):
    B, N, D = ze.shape
    K = emb_w.shape[0]
    M = B * N
    BR = 4608
    zef = ze.reshape(M, D)
    out = pl.pallas_call(
        functools.partial(_vq_body, block_rows=BR, n_codes=K, n_sub=8),
        grid=(M // BR,),
        in_specs=[pl.BlockSpec((BR, D), lambda i: (i, 0)),
                  pl.BlockSpec((K, D), lambda i: (0, 0))],
        out_specs=pl.BlockSpec((BR, D), lambda i: (i, 0)),
        out_shape=jax.ShapeDtypeStruct((M, D), jnp.float32),
    )(zef, emb_w)
    return out.reshape(B, N, D)


# speedup vs baseline: 1.5501x; 1.1491x over previous
import functools

import jax
import jax.numpy as jnp
from jax import lax
from jax.experimental import pallas as pl


def _vq_body(ze_ref, w_ref, out_ref, *, block_rows, n_codes, n_sub):
    w = w_ref[...]
    D = w.shape[1]
    w2 = lax.dot_general(
        jnp.ones((1, D), jnp.float32), w * w,
        (((1,), (1,)), ((), ())), precision=lax.Precision.HIGHEST,
        preferred_element_type=jnp.float32)
    w_hi = w.astype(jnp.bfloat16)
    w_lo = (w - w_hi.astype(jnp.float32)).astype(jnp.bfloat16)
    w_cat = jnp.concatenate((w_hi, w_lo), axis=1)
    SR = block_rows // n_sub
    iota = lax.broadcasted_iota(jnp.int32, (SR, n_codes), 1)
    for s in range(n_sub):
        zeb = ze_ref[s * SR:(s + 1) * SR, :]
        ze2 = jnp.sum(zeb * zeb, axis=1, keepdims=True)
        mm2 = lax.dot_general(2.0 * zeb, w, (((1,), (1,)), ((), ())),
                              preferred_element_type=jnp.float32)
        xs = jnp.maximum(ze2 + w2 - mm2, 1e-35)
        dist = xs * lax.rsqrt(xs)
        m = jnp.min(dist, axis=1, keepdims=True)
        idx = jnp.min(jnp.where(dist == m, iota, n_codes), axis=1,
                      keepdims=True)
        onehot = (iota == idx).astype(jnp.bfloat16)
        both = lax.dot_general(onehot, w_cat, (((1,), (0,)), ((), ())),
                               preferred_element_type=jnp.float32)
        out_ref[s * SR:(s + 1) * SR, :] = both[:, :D] + both[:, D:]


def kernel(ze, emb_w):
    B, N, D = ze.shape
    K = emb_w.shape[0]
    M = B * N
    BR = 4608
    zef = ze.reshape(M, D)
    out = pl.pallas_call(
        functools.partial(_vq_body, block_rows=BR, n_codes=K, n_sub=8),
        grid=(M // BR,),
        in_specs=[pl.BlockSpec((BR, D), lambda i: (i, 0)),
                  pl.BlockSpec((K, D), lambda i: (0, 0))],
        out_specs=pl.BlockSpec((BR, D), lambda i: (i, 0)),
        out_shape=jax.ShapeDtypeStruct((M, D), jnp.float32),
    )(zef, emb_w)
    return out.reshape(B, N, D)


# speedup vs baseline: 1.5765x; 1.0170x over previous
import functools

import jax
import jax.numpy as jnp
from jax import lax
from jax.experimental import pallas as pl


def _vq_body(ze_ref, w_ref, out_ref, *, block_rows, n_codes, n_sub):
    w = w_ref[...]
    D = w.shape[1]
    w2 = lax.dot_general(
        jnp.ones((1, D), jnp.float32), w * w,
        (((1,), (1,)), ((), ())), precision=lax.Precision.HIGHEST,
        preferred_element_type=jnp.float32)
    w_hi = w.astype(jnp.bfloat16)
    w_lo = (w - w_hi.astype(jnp.float32)).astype(jnp.bfloat16)
    w_cat = jnp.concatenate((w_hi, w_lo), axis=1)
    SR = block_rows // n_sub
    iota = lax.broadcasted_iota(jnp.int32, (SR, n_codes), 1)
    for s in range(n_sub):
        zeb = ze_ref[s * SR:(s + 1) * SR, :]
        ze2 = jnp.sum(zeb * zeb, axis=1, keepdims=True)
        mm2 = lax.dot_general(2.0 * zeb, w, (((1,), (1,)), ((), ())),
                              preferred_element_type=jnp.float32)
        xs = jnp.maximum(ze2 + w2 - mm2, 1e-35)
        dist = xs * lax.rsqrt(xs)
        m = jnp.min(dist, axis=1, keepdims=True)
        idx = jnp.min(jnp.where(dist == m, iota, n_codes), axis=1,
                      keepdims=True)
        onehot = (iota == idx).astype(jnp.bfloat16)
        both = lax.dot_general(onehot, w_cat, (((1,), (0,)), ((), ())),
                               preferred_element_type=jnp.float32)
        out_ref[s * SR:(s + 1) * SR, :] = both[:, :D] + both[:, D:]


def kernel(ze, emb_w):
    B, N, D = ze.shape
    K = emb_w.shape[0]
    M = B * N
    BR = 4608
    zef = ze.reshape(M, D)
    out = pl.pallas_call(
        functools.partial(_vq_body, block_rows=BR, n_codes=K, n_sub=6),
        grid=(M // BR,),
        in_specs=[pl.BlockSpec((BR, D), lambda i: (i, 0)),
                  pl.BlockSpec((K, D), lambda i: (0, 0))],
        out_specs=pl.BlockSpec((BR, D), lambda i: (i, 0)),
        out_shape=jax.ShapeDtypeStruct((M, D), jnp.float32),
    )(zef, emb_w)
    return out.reshape(B, N, D)
